# Initial kernel scaffold; baseline (speedup 1.0000x reference)
import jax
import jax.numpy as jnp
from jax.experimental import pallas as pl


def kernel(x, misc, tiles, coord, piece, row, col, tilecolor, zeros_param, emb_bias, W1, b1, W2, b2, W3, white_tile_mask, noise1, noise2):
    raise NotImplementedError("write your pallas kernel here")


# speedup vs baseline: 6.1634x; 6.1634x over previous
import functools

import jax
import jax.numpy as jnp
from jax import lax
from jax.experimental import pallas as pl
from jax.experimental.pallas import tpu as pltpu
from jax.experimental.pallas import tpu_sc as plsc

D = 128
B = 16384
L = 50
N_ROWS = 777

NC = 2
NS = 16
NW = NC * NS
NPW = B // NW
G = 8
GR = G * L
_CHUNKS = [(0, 104), (104, 104), (208, 104), (312, 88)]
NG = NPW // G

def _emb_pool_body(table_hbm, xflat_hbm, h_hbm, idx_v, buf_v, hout_v, gsem):
    wid = lax.axis_index("s") * NC + lax.axis_index("c")
    base = wid * NPW
    pltpu.sync_copy(xflat_hbm.at[pl.ds(base * L, NPW * L)], idx_v)

    def group_body(g, carry):
        cps = []
        for off, ln in _CHUNKS:
            cps.append(pltpu.async_copy(
                table_hbm.at[idx_v.at[pl.ds(g * GR + off, ln)]],
                buf_v.at[pl.ds(off, ln)],
                gsem,
            ))
        for cp in cps:
            cp.wait()

        def sample_body(s, c):
            accs = [jnp.zeros((16,), jnp.float32) for _ in range(D // 16)]
            for j in range(L):
                for col in range(D // 16):
                    accs[col] = accs[col] + buf_v[s * L + j, pl.ds(16 * col, 16)]
            for col in range(D // 16):
                hout_v[s, pl.ds(16 * col, 16)] = accs[col]
            return c

        lax.fori_loop(0, G, sample_body, 0, unroll=False)
        pltpu.sync_copy(hout_v, h_hbm.at[pl.ds(base + g * G, G)])
        return carry

    lax.fori_loop(0, NG, group_body, 0, unroll=False)


@functools.cache
def _emb_pool():
    mesh = plsc.VectorSubcoreMesh(core_axis_name="c", subcore_axis_name="s")
    return pl.kernel(
        _emb_pool_body,
        out_type=jax.ShapeDtypeStruct((B, D), jnp.float32),
        mesh=mesh,
        scratch_types=[
            pltpu.VMEM((NPW * L,), jnp.int32),
            pltpu.VMEM((GR, D), jnp.float32),
            pltpu.VMEM((G, D), jnp.float32),
            pltpu.SemaphoreType.DMA,
        ],
    )


_SCALE = 256.0
_QUANT = 65536.0
_LOW = _QUANT / 2.0
_SHIFT = _QUANT * 5.0 + _LOW

MLP_BLK = 2048
_NBLK = B // MLP_BLK


def _mlp_body(h_ref, n1_ref, n2_ref, eb_ref, w1_ref, b1_ref, w2_ref, b2_ref,
              w3_ref, out_ref, pen_ref):
    i = pl.program_id(0)
    h = jnp.clip(h_ref[...] + eb_ref[...], 0.0, 1.0)
    t = (lax.dot_general(h, w1_ref[...], (((1,), (1,)), ((), ())),
                         preferred_element_type=jnp.float32) + b1_ref[...]) * _SCALE
    p1 = jnp.sum((jnp.maximum(jnp.abs(t) - _LOW * 0.5, 0.0) / _SCALE) ** 2)
    t = t + n1_ref[...]
    t = ((t + _SHIFT) % _QUANT - _QUANT / 2.0) / _SCALE
    t = jnp.clip(t, 0.0, 1.0)
    t = (lax.dot_general(t, w2_ref[...], (((1,), (1,)), ((), ())),
                         preferred_element_type=jnp.float32) + b2_ref[...]) * _SCALE
    p2 = jnp.sum((jnp.maximum(jnp.abs(t) - _LOW * 0.5, 0.0) / _SCALE) ** 2)
    t = t + n2_ref[...]
    t = ((t + _SHIFT) % _QUANT - _QUANT / 2.0) / _SCALE
    t = jnp.clip(t, 0.0, 1.0)
    out_ref[...] = jnp.sum(t * w3_ref[...], axis=1, keepdims=True)

    @pl.when(i == 0)
    def _():
        pen_ref[...] = jnp.zeros_like(pen_ref)

    pen_ref[...] += jnp.reshape(p1 + p2, (1, 1))

    @pl.when(i == _NBLK - 1)
    def _():
        pen_ref[...] = pen_ref[...] * (1.0 / (B * 32.0))


_mlp = pl.pallas_call(
    _mlp_body,
    grid=(_NBLK,),
    in_specs=[
        pl.BlockSpec((MLP_BLK, D), lambda i: (i, 0)),
        pl.BlockSpec((MLP_BLK, 32), lambda i: (i, 0)),
        pl.BlockSpec((MLP_BLK, 32), lambda i: (i, 0)),
        pl.BlockSpec((1, D), lambda i: (0, 0)),
        pl.BlockSpec((32, D), lambda i: (0, 0)),
        pl.BlockSpec((1, 32), lambda i: (0, 0)),
        pl.BlockSpec((32, 32), lambda i: (0, 0)),
        pl.BlockSpec((1, 32), lambda i: (0, 0)),
        pl.BlockSpec((1, 32), lambda i: (0, 0)),
    ],
    out_specs=[
        pl.BlockSpec((MLP_BLK, 1), lambda i: (i, 0)),
        pl.BlockSpec((1, 1), lambda i: (0, 0)),
    ],
    out_shape=[
        jax.ShapeDtypeStruct((B, 1), jnp.float32),
        jax.ShapeDtypeStruct((1, 1), jnp.float32),
    ],
)


def kernel(x, misc, tiles, coord, piece, row, col, tilecolor, zeros_param,
           emb_bias, W1, b1, W2, b2, W3, white_tile_mask, noise1, noise2):
    T = (tiles + coord + piece + row + col
         + tilecolor * white_tile_mask).reshape(12 * 8 * 8, D)
    table = jnp.concatenate([T, misc, zeros_param], axis=0)
    h = _emb_pool()(table, x.reshape(-1))
    eb = (emb_bias.reshape(1, D)).astype(jnp.float32)
    out, pen = _mlp(h, noise1, noise2, eb, W1, b1.reshape(1, 32), W2,
                    b2.reshape(1, 32), W3)
    return out, pen[0, 0]


# speedup vs baseline: 9.1467x; 1.4840x over previous
import functools

import jax
import jax.numpy as jnp
from jax import lax
from jax.experimental import pallas as pl
from jax.experimental.pallas import tpu as pltpu
from jax.experimental.pallas import tpu_sc as plsc

D = 128
B = 16384
L = 50
N_ROWS = 777

NC = 2
NS = 16
NW = NC * NS
NPW = B // NW
G = 4
GR = G * L
_CHUNKS = [(0, 104), (104, 96)]
NG = NPW // G

def _emb_pool_body(table_hbm, xflat_hbm, h_hbm, idx_v,
                   buf0_v, buf1_v, hout0_v, hout1_v, sem0, sem1):
    wid = lax.axis_index("s") * NC + lax.axis_index("c")
    base = wid * NPW
    pltpu.sync_copy(xflat_hbm.at[pl.ds(base * L, NPW * L)], idx_v)

    def fire(g, buf, sem):
        for off, ln in _CHUNKS:
            pltpu.async_copy(
                table_hbm.at[idx_v.at[pl.ds(g * GR + off, ln)]],
                buf.at[pl.ds(off, ln)],
                sem,
            )

    def wait_buf(buf, sem):
        pltpu.make_async_copy(table_hbm.at[pl.ds(0, GR)], buf, sem).wait()

    def compute(g, buf, hout):
        def sample_body(s, c):
            accs = [jnp.zeros((16,), jnp.float32) for _ in range(D // 16)]
            for j in range(L):
                for col in range(D // 16):
                    accs[col] = accs[col] + buf[s * L + j, pl.ds(16 * col, 16)]
            for col in range(D // 16):
                hout[s, pl.ds(16 * col, 16)] = accs[col]
            return c

        lax.fori_loop(0, G, sample_body, 0, unroll=False)
        pltpu.sync_copy(hout, h_hbm.at[pl.ds(base + g * G, G)])

    NG2 = NG // 2
    fire(0, buf0_v, sem0)

    def pair_body(p, carry):
        g0 = 2 * p
        fire(g0 + 1, buf1_v, sem1)
        wait_buf(buf0_v, sem0)
        compute(g0, buf0_v, hout0_v)

        @pl.when(p + 1 < NG2)
        def _():
            fire(g0 + 2, buf0_v, sem0)

        wait_buf(buf1_v, sem1)
        compute(g0 + 1, buf1_v, hout1_v)
        return carry

    lax.fori_loop(0, NG2, pair_body, 0, unroll=False)


@functools.cache
def _emb_pool():
    mesh = plsc.VectorSubcoreMesh(core_axis_name="c", subcore_axis_name="s")
    return pl.kernel(
        _emb_pool_body,
        out_type=jax.ShapeDtypeStruct((B, D), jnp.float32),
        mesh=mesh,
        scratch_types=[
            pltpu.VMEM((NPW * L,), jnp.int32),
            pltpu.VMEM((GR, D), jnp.float32),
            pltpu.VMEM((GR, D), jnp.float32),
            pltpu.VMEM((G, D), jnp.float32),
            pltpu.VMEM((G, D), jnp.float32),
            pltpu.SemaphoreType.DMA,
            pltpu.SemaphoreType.DMA,
        ],
    )


_SCALE = 256.0
_QUANT = 65536.0
_LOW = _QUANT / 2.0
_SHIFT = _QUANT * 5.0 + _LOW

MLP_BLK = 2048
_NBLK = B // MLP_BLK


def _mlp_body(h_ref, n1_ref, n2_ref, eb_ref, w1_ref, b1_ref, w2_ref, b2_ref,
              w3_ref, out_ref, pen_ref):
    i = pl.program_id(0)
    h = jnp.clip(h_ref[...] + eb_ref[...], 0.0, 1.0)
    t = (lax.dot_general(h, w1_ref[...], (((1,), (1,)), ((), ())),
                         preferred_element_type=jnp.float32) + b1_ref[...]) * _SCALE
    p1 = jnp.sum((jnp.maximum(jnp.abs(t) - _LOW * 0.5, 0.0) / _SCALE) ** 2)
    t = t + n1_ref[...]
    t = ((t + _SHIFT) % _QUANT - _QUANT / 2.0) / _SCALE
    t = jnp.clip(t, 0.0, 1.0)
    t = (lax.dot_general(t, w2_ref[...], (((1,), (1,)), ((), ())),
                         preferred_element_type=jnp.float32) + b2_ref[...]) * _SCALE
    p2 = jnp.sum((jnp.maximum(jnp.abs(t) - _LOW * 0.5, 0.0) / _SCALE) ** 2)
    t = t + n2_ref[...]
    t = ((t + _SHIFT) % _QUANT - _QUANT / 2.0) / _SCALE
    t = jnp.clip(t, 0.0, 1.0)
    out_ref[...] = jnp.sum(t * w3_ref[...], axis=1, keepdims=True)

    @pl.when(i == 0)
    def _():
        pen_ref[...] = jnp.zeros_like(pen_ref)

    pen_ref[...] += jnp.reshape(p1 + p2, (1, 1))

    @pl.when(i == _NBLK - 1)
    def _():
        pen_ref[...] = pen_ref[...] * (1.0 / (B * 32.0))


_mlp = pl.pallas_call(
    _mlp_body,
    grid=(_NBLK,),
    in_specs=[
        pl.BlockSpec((MLP_BLK, D), lambda i: (i, 0)),
        pl.BlockSpec((MLP_BLK, 32), lambda i: (i, 0)),
        pl.BlockSpec((MLP_BLK, 32), lambda i: (i, 0)),
        pl.BlockSpec((1, D), lambda i: (0, 0)),
        pl.BlockSpec((32, D), lambda i: (0, 0)),
        pl.BlockSpec((1, 32), lambda i: (0, 0)),
        pl.BlockSpec((32, 32), lambda i: (0, 0)),
        pl.BlockSpec((1, 32), lambda i: (0, 0)),
        pl.BlockSpec((1, 32), lambda i: (0, 0)),
    ],
    out_specs=[
        pl.BlockSpec((MLP_BLK, 1), lambda i: (i, 0)),
        pl.BlockSpec((1, 1), lambda i: (0, 0)),
    ],
    out_shape=[
        jax.ShapeDtypeStruct((B, 1), jnp.float32),
        jax.ShapeDtypeStruct((1, 1), jnp.float32),
    ],
)


def kernel(x, misc, tiles, coord, piece, row, col, tilecolor, zeros_param,
           emb_bias, W1, b1, W2, b2, W3, white_tile_mask, noise1, noise2):
    T = (tiles + coord + piece + row + col
         + tilecolor * white_tile_mask).reshape(12 * 8 * 8, D)
    table = jnp.concatenate([T, misc, zeros_param], axis=0)
    h = _emb_pool()(table, x.reshape(-1))
    eb = (emb_bias.reshape(1, D)).astype(jnp.float32)
    out, pen = _mlp(h, noise1, noise2, eb, W1, b1.reshape(1, 32), W2,
                    b2.reshape(1, 32), W3)
    return out, pen[0, 0]
